# baseline (speedup 1.0000x reference)
import functools

import jax
import jax.numpy as jnp
from jax import lax
from jax.experimental import pallas as pl
from jax.experimental.pallas import tpu as pltpu
from jax.experimental.pallas import tpu_sc as plsc

NC = 2
NS = 16
NW = NC * NS
IDX_ROW = 128
CHUNK = 512
KG = CHUNK // IDX_ROW


def _sc_gather(table, ids_flat, n, d):
    v = table.shape[0]
    rows_per_w = n // NW
    n_chunks = rows_per_w // CHUNK
    mesh = plsc.VectorSubcoreMesh(
        core_axis_name="c", subcore_axis_name="s", num_cores=NC,
        num_subcores=NS)

    @functools.partial(
        pl.kernel,
        mesh=mesh,
        compiler_params=pltpu.CompilerParams(use_tc_tiling_on_sc=False),
        out_type=jax.ShapeDtypeStruct((n, 128), jnp.float32),
        scratch_types=[
            pltpu.VMEM((rows_per_w,), jnp.int32),
            [pltpu.VMEM((CHUNK, d), jnp.float32)] * 2,
            pltpu.VMEM_SHARED((v, d), jnp.float32),
            [pltpu.SemaphoreType.DMA] * 2,
            [pltpu.SemaphoreType.DMA] * 2,
        ],
    )
    def k(table_hbm, idx_hbm, out_hbm, idx_v, rbufs, tab_sh, gsems, wsems):
        sid = lax.axis_index("s")
        wid = sid * NC + lax.axis_index("c")

        @pl.when(sid == 0)
        def _():
            pltpu.sync_copy(table_hbm, tab_sh)

        pltpu.sync_copy(idx_hbm.at[pl.ds(wid * rows_per_w, rows_per_w)],
                        idx_v)
        plsc.subcore_barrier()
        base = wid * rows_per_w

        def fire(c, r):
            for j in range(KG):
                pltpu.async_copy(
                    tab_sh.at[idx_v.at[pl.ds(c * CHUNK + j * IDX_ROW,
                                             IDX_ROW)]],
                    rbufs[r].at[pl.ds(j * IDX_ROW, IDX_ROW)], gsems[r])

        def wait_g(r):
            for j in range(KG):
                pltpu.make_async_copy(
                    tab_sh.at[idx_v.at[pl.ds(0, IDX_ROW)]],
                    rbufs[r].at[pl.ds(j * IDX_ROW, IDX_ROW)],
                    gsems[r]).wait()

        def write(c, r):
            pltpu.async_copy(
                rbufs[r].at[:, pl.ds(0, d)],
                out_hbm.at[pl.ds(base + c * CHUNK, CHUNK), pl.ds(0, d)],
                wsems[r])

        def wait_w(r):
            pltpu.make_async_copy(
                rbufs[r].at[:, pl.ds(0, d)],
                out_hbm.at[pl.ds(0, CHUNK), pl.ds(0, d)], wsems[r]).wait()

        fire(0, 0)

        def body(c2, carry):
            for u in range(2):
                c = 2 * c2 + u
                r = u
                wait_g(r)
                write(c, r)
                rn = (u + 1) % 2

                @pl.when(c + 1 < n_chunks)
                def _():
                    @pl.when(c - 1 >= 0)
                    def _():
                        wait_w(rn)

                    fire(c + 1, rn)

            return carry

        lax.fori_loop(0, n_chunks // 2, body, 0)
        for r in range(2):
            wait_w(r)

    return k(table, ids_flat)


def kernel(token_ids, embedding_table):
    b, s = token_ids.shape
    v, d = embedding_table.shape
    n = b * s
    assert n % (NW * CHUNK) == 0
    ids_flat = token_ids.reshape(-1).astype(jnp.int32)
    out = _sc_gather(embedding_table, ids_flat, n, d)
    return out.reshape(b, s, 128)[:, :, :d]


# speedup vs baseline: 1.0071x; 1.0071x over previous
import functools

import jax
import jax.numpy as jnp
from jax import lax
from jax.experimental import pallas as pl
from jax.experimental.pallas import tpu as pltpu
from jax.experimental.pallas import tpu_sc as plsc

NC = 2
NS = 16
NW = NC * NS
NBUF = 4
FIRE_AHEAD = 2


def _sc_gather(table, ids2, b, s, d):
    v = table.shape[0]
    batches_per_w = b // NW
    mesh = plsc.VectorSubcoreMesh(
        core_axis_name="c", subcore_axis_name="s", num_cores=NC,
        num_subcores=NS)

    @functools.partial(
        pl.kernel,
        mesh=mesh,
        compiler_params=pltpu.CompilerParams(use_tc_tiling_on_sc=False),
        out_type=jax.ShapeDtypeStruct((b, s, 128), jnp.float32),
        scratch_types=[
            pltpu.VMEM((batches_per_w, s), jnp.int32),
            [pltpu.VMEM((s, d), jnp.float32)] * NBUF,
            pltpu.VMEM_SHARED((v, d), jnp.float32),
            [pltpu.SemaphoreType.DMA] * NBUF,
            [pltpu.SemaphoreType.DMA] * NBUF,
        ],
    )
    def k(table_hbm, idx_hbm, out_hbm, idx_v, rbufs, tab_sh, gsems, wsems):
        sid = lax.axis_index("s")
        wid = sid * NC + lax.axis_index("c")

        @pl.when(sid == 0)
        def _():
            pltpu.sync_copy(table_hbm, tab_sh)

        pltpu.sync_copy(idx_hbm.at[pl.ds(wid * batches_per_w,
                                         batches_per_w)], idx_v)
        plsc.subcore_barrier()

        n1 = (s // 8) * 8
        n2 = s - n1

        def fire(kk, r):
            pltpu.async_copy(
                tab_sh.at[idx_v.at[kk, pl.ds(0, n1)]],
                rbufs[r].at[pl.ds(0, n1)], gsems[r])
            if n2:
                pltpu.async_copy(
                    tab_sh.at[idx_v.at[kk, pl.ds(n1, n2)]],
                    rbufs[r].at[pl.ds(n1, n2)], gsems[r])

        def wait_g(r):
            pltpu.make_async_copy(
                tab_sh.at[idx_v.at[0, pl.ds(0, n1)]],
                rbufs[r].at[pl.ds(0, n1)], gsems[r]).wait()
            if n2:
                pltpu.make_async_copy(
                    tab_sh.at[idx_v.at[0, pl.ds(0, n2)]],
                    rbufs[r].at[pl.ds(n1, n2)], gsems[r]).wait()

        def write(kk, r):
            pltpu.async_copy(
                rbufs[r].at[:, pl.ds(0, d)],
                out_hbm.at[wid * batches_per_w + kk, :, pl.ds(0, d)],
                wsems[r])

        def wait_w(r):
            pltpu.make_async_copy(
                rbufs[r].at[:, pl.ds(0, d)],
                out_hbm.at[0, :, pl.ds(0, d)], wsems[r]).wait()

        for c in range(FIRE_AHEAD):
            fire(c, c % NBUF)

        def body(c2, carry):
            for u in range(NBUF):
                c = NBUF * c2 + u
                r = u
                wait_g(r)
                write(c, r)
                rn = (u + FIRE_AHEAD) % NBUF

                @pl.when(c + FIRE_AHEAD < batches_per_w)
                def _():
                    @pl.when(c - (NBUF - FIRE_AHEAD) >= 0)
                    def _():
                        wait_w(rn)

                    fire(c + FIRE_AHEAD, rn)

            return carry

        lax.fori_loop(0, batches_per_w // NBUF, body, 0)
        for r in range(NBUF):
            wait_w(r)

    return k(table, ids2)


def kernel(token_ids, embedding_table):
    b, s = token_ids.shape
    v, d = embedding_table.shape
    assert b % (NW * NBUF) == 0
    ids2 = token_ids.astype(jnp.int32)
    out = _sc_gather(embedding_table, ids2, b, s, d)
    return out[:, :, :d]


# speedup vs baseline: 1.0090x; 1.0020x over previous
import functools

import jax
import jax.numpy as jnp
from jax import lax
from jax.experimental import pallas as pl
from jax.experimental.pallas import tpu as pltpu
from jax.experimental.pallas import tpu_sc as plsc

NC = 2
NS = 16
NW = NC * NS
NBUF = 4
FIRE_AHEAD = 2


def _sc_gather(table, ids2, b, s, d):
    v = table.shape[0]
    batches_per_w = b // NW
    mesh = plsc.VectorSubcoreMesh(
        core_axis_name="c", subcore_axis_name="s", num_cores=NC,
        num_subcores=NS)

    @functools.partial(
        pl.kernel,
        mesh=mesh,
        compiler_params=pltpu.CompilerParams(use_tc_tiling_on_sc=False),
        out_type=jax.ShapeDtypeStruct((b, s, 128), jnp.float32),
        scratch_types=[
            pltpu.VMEM((batches_per_w, s), jnp.int32),
            [pltpu.VMEM((s, d), jnp.float32)] * NBUF,
            pltpu.VMEM_SHARED((v, d), jnp.float32),
            [pltpu.SemaphoreType.DMA] * NBUF,
            [pltpu.SemaphoreType.DMA] * NBUF,
        ],
    )
    def k(table_hbm, idx_hbm, out_hbm, idx_v, rbufs, tab_sh, gsems, wsems):
        sid = lax.axis_index("s")
        wid = sid * NC + lax.axis_index("c")

        @pl.when(sid == 0)
        def _():
            pltpu.sync_copy(table_hbm, tab_sh)

        pltpu.sync_copy(idx_hbm.at[pl.ds(wid * batches_per_w,
                                         batches_per_w)], idx_v)
        plsc.subcore_barrier()

        n1 = min(s, 128)
        n2 = s - n1

        def fire(kk, r):
            pltpu.async_copy(
                tab_sh.at[idx_v.at[kk, pl.ds(0, n1)]],
                rbufs[r].at[pl.ds(0, n1)], gsems[r])
            if n2:
                pltpu.async_copy(
                    tab_sh.at[idx_v.at[kk, pl.ds(n1, n2)]],
                    rbufs[r].at[pl.ds(n1, n2)], gsems[r])

        def wait_g(r):
            pltpu.make_async_copy(
                tab_sh.at[idx_v.at[0, pl.ds(0, n1)]],
                rbufs[r].at[pl.ds(0, n1)], gsems[r]).wait()
            if n2:
                pltpu.make_async_copy(
                    tab_sh.at[idx_v.at[0, pl.ds(0, n2)]],
                    rbufs[r].at[pl.ds(n1, n2)], gsems[r]).wait()

        def write(kk, r):
            pltpu.async_copy(
                rbufs[r].at[:, pl.ds(0, d)],
                out_hbm.at[wid * batches_per_w + kk, :, pl.ds(0, d)],
                wsems[r])

        def wait_w(r):
            pltpu.make_async_copy(
                rbufs[r].at[:, pl.ds(0, d)],
                out_hbm.at[0, :, pl.ds(0, d)], wsems[r]).wait()

        for c in range(FIRE_AHEAD):
            fire(c, c % NBUF)

        def body(c2, carry):
            for u in range(NBUF):
                c = NBUF * c2 + u
                r = u
                wait_g(r)
                write(c, r)
                rn = (u + FIRE_AHEAD) % NBUF

                @pl.when(c + FIRE_AHEAD < batches_per_w)
                def _():
                    @pl.when(c - (NBUF - FIRE_AHEAD) >= 0)
                    def _():
                        wait_w(rn)

                    fire(c + FIRE_AHEAD, rn)

            return carry

        lax.fori_loop(0, batches_per_w // NBUF, body, 0)
        for r in range(NBUF):
            wait_w(r)

    return k(table, ids2)


def kernel(token_ids, embedding_table):
    b, s = token_ids.shape
    v, d = embedding_table.shape
    assert b % (NW * NBUF) == 0
    ids2 = token_ids.astype(jnp.int32)
    out = _sc_gather(embedding_table, ids2, b, s, d)
    return out[:, :, :d]
